# Initial kernel scaffold; baseline (speedup 1.0000x reference)
import jax
import jax.numpy as jnp
from jax.experimental import pallas as pl


def kernel(x, edge_index, W1, b1, W2, b2):
    raise NotImplementedError("write your pallas kernel here")


# speedup vs baseline: 26.3648x; 26.3648x over previous
import functools

import jax
import jax.numpy as jnp
from jax import lax
from jax.experimental import pallas as pl
from jax.experimental.pallas import tpu as pltpu
from jax.experimental.pallas import tpu_sc as plsc

N = 10000
NP = 10240
E = 320000
NC = 2
NS = 16
NW = NC * NS
EPW = E // NW
C = 80
K = EPW // C
RPT = NP // NS

_mesh = plsc.VectorSubcoreMesh(core_axis_name="c", subcore_axis_name="s")


def _make_edge_pass(width):
  @functools.partial(
      pl.kernel,
      mesh=_mesh,
      out_type=jax.ShapeDtypeStruct((NC, NP, width), jnp.float32),
      scratch_types=[
          pltpu.VMEM((K, C), jnp.int32),
          pltpu.VMEM((K, C), jnp.int32),
          pltpu.VMEM((C, width), jnp.float32),
          pltpu.VMEM_SHARED((NP, width), jnp.float32),
          pltpu.SemaphoreType.DMA,
      ],
      compiler_params=pltpu.CompilerParams(use_tc_tiling_on_sc=False),
  )
  def edge_pass(y_hbm, src_hbm, dst_hbm, z_hbm, o_hbm,
                dst_v, src_v, rows_v, acc, sem):
    c = lax.axis_index("c")
    s = lax.axis_index("s")
    w = c * NS + s
    pltpu.sync_copy(dst_hbm.at[w], dst_v)
    pltpu.sync_copy(src_hbm.at[w], src_v)
    r0 = s * RPT
    pltpu.sync_copy(z_hbm.at[pl.ds(r0, RPT)], acc.at[pl.ds(r0, RPT)])
    plsc.subcore_barrier()

    @pl.loop(0, K)
    def _(j):
      pltpu.async_copy(y_hbm.at[src_v.at[j]], rows_v, sem).wait()
      pltpu.sync_copy(rows_v, acc.at[dst_v.at[j]], add=True)

    plsc.subcore_barrier()
    pltpu.sync_copy(acc.at[pl.ds(r0, RPT)], o_hbm.at[c, pl.ds(r0, RPT)])

  return edge_pass


_edge64 = _make_edge_pass(64)
_edge16 = _make_edge_pass(16)


@functools.partial(
    pl.kernel,
    mesh=_mesh,
    out_type=jax.ShapeDtypeStruct((NC, NP, 16), jnp.float32),
    scratch_types=[
        pltpu.VMEM((K, C), jnp.int32),
        pltpu.VMEM((C, 16), jnp.float32),
        pltpu.VMEM_SHARED((NP, 16), jnp.float32),
        pltpu.SemaphoreType.DMA,
    ],
    compiler_params=pltpu.CompilerParams(use_tc_tiling_on_sc=False),
)
def _deg_pass(ones_hbm, dst_hbm, z_hbm, o_hbm, dst_v, ones_v, acc, sem):
  c = lax.axis_index("c")
  s = lax.axis_index("s")
  w = c * NS + s
  pltpu.sync_copy(dst_hbm.at[w], dst_v)
  pltpu.sync_copy(ones_hbm, ones_v)
  r0 = s * RPT
  pltpu.sync_copy(z_hbm.at[pl.ds(r0, RPT)], acc.at[pl.ds(r0, RPT)])
  plsc.subcore_barrier()

  @pl.loop(0, K)
  def _(j):
    pltpu.sync_copy(ones_v, acc.at[dst_v.at[j]], add=True)

  plsc.subcore_barrier()
  pltpu.sync_copy(acc.at[pl.ds(r0, RPT)], o_hbm.at[c, pl.ds(r0, RPT)])


_BLK = 1000
_GRID = N // _BLK


def _matmul_xw(x, W1):
  def body(x_ref, w_ref, o_ref):
    o_ref[...] = jnp.dot(x_ref[...], w_ref[...],
                         preferred_element_type=jnp.float32)

  return pl.pallas_call(
      body,
      grid=(_GRID,),
      in_specs=[
          pl.BlockSpec((_BLK, 128), lambda i: (i, 0)),
          pl.BlockSpec((128, 64), lambda i: (0, 0)),
      ],
      out_specs=pl.BlockSpec((_BLK, 64), lambda i: (i, 0)),
      out_shape=jax.ShapeDtypeStruct((N, 64), jnp.float32),
  )(x, W1)


def _scale_y1(dp, xw):
  def body(dp_ref, xw_ref, o_ref):
    deg = dp_ref[0, :, 0:1] + dp_ref[1, :, 0:1] + 1.0
    o_ref[...] = lax.rsqrt(deg) * xw_ref[...]

  return pl.pallas_call(
      body,
      grid=(_GRID,),
      in_specs=[
          pl.BlockSpec((2, _BLK, 16), lambda i: (0, i, 0)),
          pl.BlockSpec((_BLK, 64), lambda i: (i, 0)),
      ],
      out_specs=pl.BlockSpec((_BLK, 64), lambda i: (i, 0)),
      out_shape=jax.ShapeDtypeStruct((N, 64), jnp.float32),
  )(dp, xw)


def _layer2_prep(dp, p, y1, b1r, W2p):
  def body(dp_ref, p_ref, y1_ref, b1_ref, w2_ref, o_ref):
    deg = dp_ref[0, :, 0:1] + dp_ref[1, :, 0:1] + 1.0
    dis = lax.rsqrt(deg)
    h = jnp.maximum(
        dis * (p_ref[0] + p_ref[1] + y1_ref[...]) + b1_ref[...], 0.0)
    o_ref[...] = dis * jnp.dot(h, w2_ref[...],
                               preferred_element_type=jnp.float32)

  return pl.pallas_call(
      body,
      grid=(_GRID,),
      in_specs=[
          pl.BlockSpec((2, _BLK, 16), lambda i: (0, i, 0)),
          pl.BlockSpec((2, _BLK, 64), lambda i: (0, i, 0)),
          pl.BlockSpec((_BLK, 64), lambda i: (i, 0)),
          pl.BlockSpec((1, 64), lambda i: (0, 0)),
          pl.BlockSpec((64, 16), lambda i: (0, 0)),
      ],
      out_specs=pl.BlockSpec((_BLK, 16), lambda i: (i, 0)),
      out_shape=jax.ShapeDtypeStruct((N, 16), jnp.float32),
  )(dp, p, y1, b1r, W2p)


def _final(dp, q, y2p, b2p):
  def body(dp_ref, q_ref, y2_ref, b2_ref, o_ref):
    deg = dp_ref[0, :, 0:1] + dp_ref[1, :, 0:1] + 1.0
    dis = lax.rsqrt(deg)
    o_ref[...] = dis * (q_ref[0] + q_ref[1] + y2_ref[...]) + b2_ref[...]

  return pl.pallas_call(
      body,
      grid=(_GRID,),
      in_specs=[
          pl.BlockSpec((2, _BLK, 16), lambda i: (0, i, 0)),
          pl.BlockSpec((2, _BLK, 16), lambda i: (0, i, 0)),
          pl.BlockSpec((_BLK, 16), lambda i: (i, 0)),
          pl.BlockSpec((1, 16), lambda i: (0, 0)),
      ],
      out_specs=pl.BlockSpec((_BLK, 16), lambda i: (i, 0)),
      out_shape=jax.ShapeDtypeStruct((N, 16), jnp.float32),
  )(dp, q, y2p, b2p)


def kernel(x, edge_index, W1, b1, W2, b2):
  x = x.astype(jnp.float32)
  src3 = edge_index[0].reshape(NW, K, C)
  dst3 = edge_index[1].reshape(NW, K, C)
  zeros16 = jnp.zeros((NP, 16), jnp.float32)
  zeros64 = jnp.zeros((NP, 64), jnp.float32)
  ones = jnp.ones((C, 16), jnp.float32)
  W2p = jnp.pad(W2, ((0, 0), (0, 14)))
  b1r = b1.reshape(1, 64)
  b2p = jnp.pad(b2, (0, 14)).reshape(1, 16)

  dp = _deg_pass(ones, dst3, zeros16)
  xw = _matmul_xw(x, W1)
  y1 = _scale_y1(dp, xw)
  p = _edge64(y1, src3, dst3, zeros64)
  y2p = _layer2_prep(dp, p, y1, b1r, W2p)
  q = _edge16(y2p, src3, dst3, zeros16)
  outp = _final(dp, q, y2p, b2p)
  return outp[:, :2]


# speedup vs baseline: 47.2277x; 1.7913x over previous
import functools

import jax
import jax.numpy as jnp
from jax import lax
from jax.experimental import pallas as pl
from jax.experimental.pallas import tpu as pltpu
from jax.experimental.pallas import tpu_sc as plsc

N = 10000
NP = 10240
E = 320000
NC = 2
NS = 16
NW = NC * NS
EPW = E // NW
C = 80
K = EPW // C
RPT = NP // NS

_mesh = plsc.VectorSubcoreMesh(core_axis_name="c", subcore_axis_name="s")


NBUF = 5


def _make_edge_pass(width):
  @functools.partial(
      pl.kernel,
      mesh=_mesh,
      out_type=jax.ShapeDtypeStruct((NC, NP, width), jnp.float32),
      scratch_types=[
          pltpu.VMEM((K, C), jnp.int32),
          pltpu.VMEM((K, C), jnp.int32),
      ] + [pltpu.VMEM((C, width), jnp.float32) for _ in range(NBUF)]
        + [pltpu.VMEM_SHARED((NP, width), jnp.float32)]
        + [pltpu.SemaphoreType.DMA for _ in range(NBUF)],
      compiler_params=pltpu.CompilerParams(use_tc_tiling_on_sc=False),
  )
  def edge_pass(y_hbm, src_hbm, dst_hbm, z_hbm, o_hbm,
                dst_v, src_v, *rest):
    rows = rest[:NBUF]
    acc = rest[NBUF]
    gsem = rest[NBUF + 1:]
    c = lax.axis_index("c")
    s = lax.axis_index("s")
    w = c * NS + s
    pltpu.sync_copy(dst_hbm.at[w], dst_v)
    pltpu.sync_copy(src_hbm.at[w], src_v)
    r0 = s * RPT
    pltpu.sync_copy(z_hbm.at[pl.ds(r0, RPT)], acc.at[pl.ds(r0, RPT)])
    plsc.subcore_barrier()

    for b in range(NBUF):
      pltpu.async_copy(y_hbm.at[src_v.at[b]], rows[b], gsem[b])

    @pl.loop(0, K // NBUF)
    def _(g):
      base = g * NBUF
      for b in range(NBUF):
        j = base + b
        pltpu.make_async_copy(y_hbm.at[src_v.at[j]], rows[b], gsem[b]).wait()
        pltpu.sync_copy(rows[b], acc.at[dst_v.at[j]], add=True)
        nxt = j + NBUF

        @pl.when(nxt < K)
        def _():
          pltpu.async_copy(y_hbm.at[src_v.at[nxt]], rows[b], gsem[b])

    plsc.subcore_barrier()
    pltpu.sync_copy(acc.at[pl.ds(r0, RPT)], o_hbm.at[c, pl.ds(r0, RPT)])

  return edge_pass


_edge64 = _make_edge_pass(64)
_edge16 = _make_edge_pass(16)


@functools.partial(
    pl.kernel,
    mesh=_mesh,
    out_type=jax.ShapeDtypeStruct((NC, NP, 16), jnp.float32),
    scratch_types=[
        pltpu.VMEM((K, C), jnp.int32),
        pltpu.VMEM((C, 16), jnp.float32),
        pltpu.VMEM_SHARED((NP, 16), jnp.float32),
        pltpu.SemaphoreType.DMA,
    ],
    compiler_params=pltpu.CompilerParams(use_tc_tiling_on_sc=False),
)
def _deg_pass(ones_hbm, dst_hbm, z_hbm, o_hbm, dst_v, ones_v, acc, sem):
  c = lax.axis_index("c")
  s = lax.axis_index("s")
  w = c * NS + s
  pltpu.sync_copy(dst_hbm.at[w], dst_v)
  pltpu.sync_copy(ones_hbm, ones_v)
  r0 = s * RPT
  pltpu.sync_copy(z_hbm.at[pl.ds(r0, RPT)], acc.at[pl.ds(r0, RPT)])
  plsc.subcore_barrier()

  @pl.loop(0, K)
  def _(j):
    pltpu.sync_copy(ones_v, acc.at[dst_v.at[j]], add=True)

  plsc.subcore_barrier()
  pltpu.sync_copy(acc.at[pl.ds(r0, RPT)], o_hbm.at[c, pl.ds(r0, RPT)])


_BLK = 1000
_GRID = N // _BLK


def _matmul_xw(x, W1):
  def body(x_ref, w_ref, o_ref):
    o_ref[...] = jnp.dot(x_ref[...], w_ref[...],
                         preferred_element_type=jnp.float32)

  return pl.pallas_call(
      body,
      grid=(_GRID,),
      in_specs=[
          pl.BlockSpec((_BLK, 128), lambda i: (i, 0)),
          pl.BlockSpec((128, 64), lambda i: (0, 0)),
      ],
      out_specs=pl.BlockSpec((_BLK, 64), lambda i: (i, 0)),
      out_shape=jax.ShapeDtypeStruct((N, 64), jnp.float32),
  )(x, W1)


def _scale_y1(dp, xw):
  def body(dp_ref, xw_ref, o_ref):
    deg = dp_ref[0, :, 0:1] + dp_ref[1, :, 0:1] + 1.0
    o_ref[...] = lax.rsqrt(deg) * xw_ref[...]

  return pl.pallas_call(
      body,
      grid=(_GRID,),
      in_specs=[
          pl.BlockSpec((2, _BLK, 16), lambda i: (0, i, 0)),
          pl.BlockSpec((_BLK, 64), lambda i: (i, 0)),
      ],
      out_specs=pl.BlockSpec((_BLK, 64), lambda i: (i, 0)),
      out_shape=jax.ShapeDtypeStruct((N, 64), jnp.float32),
  )(dp, xw)


def _layer2_prep(dp, p, y1, b1r, W2p):
  def body(dp_ref, p_ref, y1_ref, b1_ref, w2_ref, o_ref):
    deg = dp_ref[0, :, 0:1] + dp_ref[1, :, 0:1] + 1.0
    dis = lax.rsqrt(deg)
    h = jnp.maximum(
        dis * (p_ref[0] + p_ref[1] + y1_ref[...]) + b1_ref[...], 0.0)
    o_ref[...] = dis * jnp.dot(h, w2_ref[...],
                               preferred_element_type=jnp.float32)

  return pl.pallas_call(
      body,
      grid=(_GRID,),
      in_specs=[
          pl.BlockSpec((2, _BLK, 16), lambda i: (0, i, 0)),
          pl.BlockSpec((2, _BLK, 64), lambda i: (0, i, 0)),
          pl.BlockSpec((_BLK, 64), lambda i: (i, 0)),
          pl.BlockSpec((1, 64), lambda i: (0, 0)),
          pl.BlockSpec((64, 16), lambda i: (0, 0)),
      ],
      out_specs=pl.BlockSpec((_BLK, 16), lambda i: (i, 0)),
      out_shape=jax.ShapeDtypeStruct((N, 16), jnp.float32),
  )(dp, p, y1, b1r, W2p)


def _final(dp, q, y2p, b2p):
  def body(dp_ref, q_ref, y2_ref, b2_ref, o_ref):
    deg = dp_ref[0, :, 0:1] + dp_ref[1, :, 0:1] + 1.0
    dis = lax.rsqrt(deg)
    o_ref[...] = dis * (q_ref[0] + q_ref[1] + y2_ref[...]) + b2_ref[...]

  return pl.pallas_call(
      body,
      grid=(_GRID,),
      in_specs=[
          pl.BlockSpec((2, _BLK, 16), lambda i: (0, i, 0)),
          pl.BlockSpec((2, _BLK, 16), lambda i: (0, i, 0)),
          pl.BlockSpec((_BLK, 16), lambda i: (i, 0)),
          pl.BlockSpec((1, 16), lambda i: (0, 0)),
      ],
      out_specs=pl.BlockSpec((_BLK, 16), lambda i: (i, 0)),
      out_shape=jax.ShapeDtypeStruct((N, 16), jnp.float32),
  )(dp, q, y2p, b2p)


def kernel(x, edge_index, W1, b1, W2, b2):
  x = x.astype(jnp.float32)
  src3 = edge_index[0].reshape(NW, K, C)
  dst3 = edge_index[1].reshape(NW, K, C)
  zeros16 = jnp.zeros((NP, 16), jnp.float32)
  zeros64 = jnp.zeros((NP, 64), jnp.float32)
  ones = jnp.ones((C, 16), jnp.float32)
  W2p = jnp.pad(W2, ((0, 0), (0, 14)))
  b1r = b1.reshape(1, 64)
  b2p = jnp.pad(b2, (0, 14)).reshape(1, 16)

  dp = _deg_pass(ones, dst3, zeros16)
  xw = _matmul_xw(x, W1)
  y1 = _scale_y1(dp, xw)
  p = _edge64(y1, src3, dst3, zeros64)
  y2p = _layer2_prep(dp, p, y1, b1r, W2p)
  q = _edge16(y2p, src3, dst3, zeros16)
  outp = _final(dp, q, y2p, b2p)
  return outp[:, :2]


# speedup vs baseline: 48.3652x; 1.0241x over previous
import functools

import jax
import jax.numpy as jnp
from jax import lax
from jax.experimental import pallas as pl
from jax.experimental.pallas import tpu as pltpu
from jax.experimental.pallas import tpu_sc as plsc

N = 10000
NP = 10240
E = 320000
NC = 2
NS = 16
NW = NC * NS
EPW = E // NW
C = 80
K = EPW // C
RPT = NP // NS

_mesh = plsc.VectorSubcoreMesh(core_axis_name="c", subcore_axis_name="s")


NBUF = 10
HALF = NBUF // 2
PAIRS = K // NBUF


def _make_edge_pass(width):
  @functools.partial(
      pl.kernel,
      mesh=_mesh,
      out_type=jax.ShapeDtypeStruct((NC, NP, width), jnp.float32),
      scratch_types=[
          pltpu.VMEM((K, C), jnp.int32),
          pltpu.VMEM((K, C), jnp.int32),
      ] + [pltpu.VMEM((C, width), jnp.float32) for _ in range(NBUF)]
        + [pltpu.VMEM_SHARED((NP, width), jnp.float32)]
        + [pltpu.SemaphoreType.DMA for _ in range(2 * NBUF)],
      compiler_params=pltpu.CompilerParams(use_tc_tiling_on_sc=False),
  )
  def edge_pass(y_hbm, src_hbm, dst_hbm, z_hbm, o_hbm,
                dst_v, src_v, *rest):
    rows = rest[:NBUF]
    acc = rest[NBUF]
    gsem = rest[NBUF + 1:NBUF + 1 + NBUF]
    ssem = rest[NBUF + 1 + NBUF:]

    class _Op:
      def __init__(self, src, dst, sem, add=False):
        self.args, self.add = (src, dst, sem), add

      def start(self):
        pltpu.async_copy(*self.args, add=self.add)

      def wait(self):
        pltpu.make_async_copy(*self.args).wait()

    def gath(j, b):
      return _Op(y_hbm.at[src_v.at[j]], rows[b], gsem[b])

    def scat(j, b):
      return _Op(rows[b], acc.at[dst_v.at[j]], ssem[b], add=True)

    c = lax.axis_index("c")
    s = lax.axis_index("s")
    w = c * NS + s
    pltpu.sync_copy(dst_hbm.at[w], dst_v)
    pltpu.sync_copy(src_hbm.at[w], src_v)
    r0 = s * RPT
    pltpu.sync_copy(z_hbm.at[pl.ds(r0, RPT)], acc.at[pl.ds(r0, RPT)])
    plsc.subcore_barrier()

    for b in range(HALF):
      gath(b, b).start()

    @pl.loop(0, PAIRS)
    def _(r):
      j0 = r * NBUF
      for b in range(NBUF):
        j = j0 + b
        gath(j, b).wait()
        scat(j, b).start()
        bn = (b + HALF) % NBUF
        if b < HALF:
          @pl.when(r > 0)
          def _():
            scat(j - HALF, bn).wait()
        else:
          scat(j - HALF, bn).wait()
        gath(j + HALF, bn).start()

    for b in range(HALF):
      j = PAIRS * NBUF + b
      gath(j, b).wait()
      scat(j, b).start()
    for b in range(HALF):
      scat(PAIRS * NBUF + b, b).wait()
      scat(PAIRS * NBUF - HALF + b, b + HALF).wait()

    plsc.subcore_barrier()
    pltpu.sync_copy(acc.at[pl.ds(r0, RPT)], o_hbm.at[c, pl.ds(r0, RPT)])

  return edge_pass


_edge64 = _make_edge_pass(64)
_edge16 = _make_edge_pass(16)


@functools.partial(
    pl.kernel,
    mesh=_mesh,
    out_type=jax.ShapeDtypeStruct((NC, NP, 16), jnp.float32),
    scratch_types=[
        pltpu.VMEM((K, C), jnp.int32),
        pltpu.VMEM((C, 16), jnp.float32),
        pltpu.VMEM_SHARED((NP, 16), jnp.float32),
        pltpu.SemaphoreType.DMA,
    ],
    compiler_params=pltpu.CompilerParams(use_tc_tiling_on_sc=False),
)
def _deg_pass(ones_hbm, dst_hbm, z_hbm, o_hbm, dst_v, ones_v, acc, sem):
  c = lax.axis_index("c")
  s = lax.axis_index("s")
  w = c * NS + s
  pltpu.sync_copy(dst_hbm.at[w], dst_v)
  pltpu.sync_copy(ones_hbm, ones_v)
  r0 = s * RPT
  pltpu.sync_copy(z_hbm.at[pl.ds(r0, RPT)], acc.at[pl.ds(r0, RPT)])
  plsc.subcore_barrier()

  LAG = 8

  @pl.loop(0, K)
  def _(j):
    pltpu.async_copy(ones_v, acc.at[dst_v.at[j]], sem, add=True)

    @pl.when(j >= LAG)
    def _():
      pltpu.make_async_copy(ones_v, acc.at[dst_v.at[j - LAG]], sem).wait()

  @pl.loop(K - LAG, K)
  def _(j):
    pltpu.make_async_copy(ones_v, acc.at[dst_v.at[j]], sem).wait()

  plsc.subcore_barrier()
  pltpu.sync_copy(acc.at[pl.ds(r0, RPT)], o_hbm.at[c, pl.ds(r0, RPT)])


_BLK = 1000
_GRID = N // _BLK


def _matmul_xw(x, W1):
  def body(x_ref, w_ref, o_ref):
    o_ref[...] = jnp.dot(x_ref[...], w_ref[...],
                         preferred_element_type=jnp.float32)

  return pl.pallas_call(
      body,
      grid=(_GRID,),
      in_specs=[
          pl.BlockSpec((_BLK, 128), lambda i: (i, 0)),
          pl.BlockSpec((128, 64), lambda i: (0, 0)),
      ],
      out_specs=pl.BlockSpec((_BLK, 64), lambda i: (i, 0)),
      out_shape=jax.ShapeDtypeStruct((N, 64), jnp.float32),
  )(x, W1)


def _scale_y1(dp, xw):
  def body(dp_ref, xw_ref, o_ref):
    deg = dp_ref[0, :, 0:1] + dp_ref[1, :, 0:1] + 1.0
    o_ref[...] = lax.rsqrt(deg) * xw_ref[...]

  return pl.pallas_call(
      body,
      grid=(_GRID,),
      in_specs=[
          pl.BlockSpec((2, _BLK, 16), lambda i: (0, i, 0)),
          pl.BlockSpec((_BLK, 64), lambda i: (i, 0)),
      ],
      out_specs=pl.BlockSpec((_BLK, 64), lambda i: (i, 0)),
      out_shape=jax.ShapeDtypeStruct((N, 64), jnp.float32),
  )(dp, xw)


def _layer2_prep(dp, p, y1, b1r, W2p):
  def body(dp_ref, p_ref, y1_ref, b1_ref, w2_ref, o_ref):
    deg = dp_ref[0, :, 0:1] + dp_ref[1, :, 0:1] + 1.0
    dis = lax.rsqrt(deg)
    h = jnp.maximum(
        dis * (p_ref[0] + p_ref[1] + y1_ref[...]) + b1_ref[...], 0.0)
    o_ref[...] = dis * jnp.dot(h, w2_ref[...],
                               preferred_element_type=jnp.float32)

  return pl.pallas_call(
      body,
      grid=(_GRID,),
      in_specs=[
          pl.BlockSpec((2, _BLK, 16), lambda i: (0, i, 0)),
          pl.BlockSpec((2, _BLK, 64), lambda i: (0, i, 0)),
          pl.BlockSpec((_BLK, 64), lambda i: (i, 0)),
          pl.BlockSpec((1, 64), lambda i: (0, 0)),
          pl.BlockSpec((64, 16), lambda i: (0, 0)),
      ],
      out_specs=pl.BlockSpec((_BLK, 16), lambda i: (i, 0)),
      out_shape=jax.ShapeDtypeStruct((N, 16), jnp.float32),
  )(dp, p, y1, b1r, W2p)


def _final(dp, q, y2p, b2p):
  def body(dp_ref, q_ref, y2_ref, b2_ref, o_ref):
    deg = dp_ref[0, :, 0:1] + dp_ref[1, :, 0:1] + 1.0
    dis = lax.rsqrt(deg)
    o_ref[...] = dis * (q_ref[0] + q_ref[1] + y2_ref[...]) + b2_ref[...]

  return pl.pallas_call(
      body,
      grid=(_GRID,),
      in_specs=[
          pl.BlockSpec((2, _BLK, 16), lambda i: (0, i, 0)),
          pl.BlockSpec((2, _BLK, 16), lambda i: (0, i, 0)),
          pl.BlockSpec((_BLK, 16), lambda i: (i, 0)),
          pl.BlockSpec((1, 16), lambda i: (0, 0)),
      ],
      out_specs=pl.BlockSpec((_BLK, 16), lambda i: (i, 0)),
      out_shape=jax.ShapeDtypeStruct((N, 16), jnp.float32),
  )(dp, q, y2p, b2p)


def kernel(x, edge_index, W1, b1, W2, b2):
  x = x.astype(jnp.float32)
  src3 = edge_index[0].reshape(NW, K, C)
  dst3 = edge_index[1].reshape(NW, K, C)
  zeros16 = jnp.zeros((NP, 16), jnp.float32)
  zeros64 = jnp.zeros((NP, 64), jnp.float32)
  ones = jnp.ones((C, 16), jnp.float32)
  W2p = jnp.pad(W2, ((0, 0), (0, 14)))
  b1r = b1.reshape(1, 64)
  b2p = jnp.pad(b2, (0, 14)).reshape(1, 16)

  dp = _deg_pass(ones, dst3, zeros16)
  xw = _matmul_xw(x, W1)
  y1 = _scale_y1(dp, xw)
  p = _edge64(y1, src3, dst3, zeros64)
  y2p = _layer2_prep(dp, p, y1, b1r, W2p)
  q = _edge16(y2p, src3, dst3, zeros16)
  outp = _final(dp, q, y2p, b2p)
  return outp[:, :2]
